# Initial kernel scaffold; baseline (speedup 1.0000x reference)
import jax
import jax.numpy as jnp
from jax.experimental import pallas as pl


def kernel(x, graph, W, b):
    raise NotImplementedError("write your pallas kernel here")


# speedup vs baseline: 2028.8273x; 2028.8273x over previous
import functools

import jax
import jax.numpy as jnp
from jax.experimental import pallas as pl


def _gcn_body(x_ref, graph_ref, w_ref, b_ref, out_ref, *, chunk):
    g = graph_ref[...].astype(jnp.float32)
    L = g.shape[0]
    deg = jnp.sum(g, axis=0) + 1.0
    dinv = jax.lax.rsqrt(deg)
    s = (g.T + jnp.eye(L, dtype=jnp.float32)) * (dinv[:, None] * dinv[None, :])

    xb = x_ref[...]
    f = xb.shape[-1]
    xw = jnp.dot(xb.reshape(chunk * L, f), w_ref[...],
                 preferred_element_type=jnp.float32).reshape(chunk, L, f)
    sb = jnp.broadcast_to(s, (chunk, L, L))
    agg = jax.lax.dot_general(sb, xw, (((2,), (1,)), ((0,), (0,))),
                              preferred_element_type=jnp.float32)
    out_ref[...] = jnp.maximum(agg + b_ref[...], 0.0) + xb


def kernel(x, graph, W, b):
    bsz, len_, d = x.shape
    chunk = 16
    grid = (bsz // chunk,)
    out = pl.pallas_call(
        functools.partial(_gcn_body, chunk=chunk),
        grid=grid,
        in_specs=[
            pl.BlockSpec((chunk, len_, d), lambda i: (i, 0, 0)),
            pl.BlockSpec((len_, len_), lambda i: (0, 0)),
            pl.BlockSpec((d, d), lambda i: (0, 0)),
            pl.BlockSpec((1, d), lambda i: (0, 0)),
        ],
        out_specs=pl.BlockSpec((chunk, len_, d), lambda i: (i, 0, 0)),
        out_shape=jax.ShapeDtypeStruct((bsz, len_, d), x.dtype),
    )(x, graph, W, b.reshape(1, d))
    return out
